# baseline (speedup 1.0000x reference)
import functools

import jax
import jax.numpy as jnp
from jax import lax
from jax.experimental import pallas as pl
from jax.experimental.pallas import tpu as pltpu
from jax.experimental.pallas import tpu_sc as plsc

NUM_EMBEDDINGS = 1000000
EMBED_DIM = 32
BATCH = 16384
HIST = 50

LANE = 128
TOTAL = BATCH * HIST
NROWS = TOTAL // LANE
NW = 32
ROWS_PER_W = NROWS // NW
KK = 25
NGRP = ROWS_PER_W // KK


def _build_kernel():
    mesh = plsc.VectorSubcoreMesh(core_axis_name="c", subcore_axis_name="s")

    @functools.partial(
        pl.kernel,
        mesh=mesh,
        out_type=jax.ShapeDtypeStruct((NROWS, LANE, EMBED_DIM), jnp.float32),
        scratch_types=[
            pltpu.VMEM((ROWS_PER_W, LANE), jnp.int32),
            pltpu.VMEM((KK, LANE, EMBED_DIM), jnp.float32),
            pltpu.SemaphoreType.DMA,
        ],
        compiler_params=pltpu.CompilerParams(use_tc_tiling_on_sc=False),
    )
    def gather_kernel(idx_hbm, table_hbm, out_hbm, idx_v, rows_v, gsem):
        wid = lax.axis_index("s") * 2 + lax.axis_index("c")
        base = wid * ROWS_PER_W
        pltpu.sync_copy(idx_hbm.at[pl.ds(base, ROWS_PER_W)], idx_v)

        def step(g, carry):
            for j in range(KK):
                pltpu.async_copy(
                    table_hbm.at[idx_v.at[g * KK + j]], rows_v.at[j], gsem
                )
            for j in range(KK):
                pltpu.make_async_copy(
                    table_hbm.at[idx_v.at[j]], rows_v.at[j], gsem
                ).wait()
            pltpu.sync_copy(rows_v, out_hbm.at[pl.ds(base + g * KK, KK)])
            return carry

        lax.fori_loop(0, NGRP, step, 0)

    return gather_kernel


_gather = _build_kernel()


@jax.jit
def kernel(inputs, embedding):
    idx = inputs.astype(jnp.int32).reshape(NROWS, LANE)
    out = _gather(idx, embedding)
    return out.reshape(BATCH, HIST, EMBED_DIM)
